# Initial kernel scaffold; baseline (speedup 1.0000x reference)
import jax
import jax.numpy as jnp
from jax.experimental import pallas as pl


def kernel(x, edge_index, edge_weight):
    raise NotImplementedError("write your pallas kernel here")


# speedup vs baseline: 3.7637x; 3.7637x over previous
import functools

import jax
import jax.numpy as jnp
from jax import lax
from jax.experimental import pallas as pl
from jax.experimental.pallas import tpu as pltpu
from jax.experimental.pallas import tpu_sc as plsc

NC = 2
NS = 16
L = 16
NW = NC * NS
EDGE_BATCH = 128


@functools.lru_cache(maxsize=None)
def _build_sc_kernel(n_nodes, d_feat, e_pad):
  assert n_nodes % NS == 0 and d_feat % L == 0
  assert e_pad % (NW * EDGE_BATCH) == 0
  epw = e_pad // NW
  n_batches = epw // EDGE_BATCH
  zt = 10
  rows_per_tile = n_nodes // zt
  assert rows_per_tile % 8 == 0
  zr = 200 if rows_per_tile % 200 == 0 else rows_per_tile
  n_chunks = d_feat // L

  mesh = plsc.VectorSubcoreMesh(core_axis_name="c", subcore_axis_name="s",
                                num_cores=NC)

  @functools.partial(
      pl.kernel,
      out_type=jax.ShapeDtypeStruct((NC, n_nodes, d_feat), jnp.float32),
      mesh=mesh,
      scratch_types=[
          pltpu.VMEM_SHARED((n_nodes, d_feat), jnp.float32),
          pltpu.VMEM((EDGE_BATCH,), jnp.int32),
          pltpu.VMEM((EDGE_BATCH,), jnp.int32),
          pltpu.VMEM((EDGE_BATCH,), jnp.float32),
          pltpu.VMEM((EDGE_BATCH, d_feat), jnp.float32),
          pltpu.VMEM((zr, d_feat), jnp.float32),
          pltpu.SemaphoreType.DMA,
      ],
  )
  def sc_kernel(x_hbm, src_hbm, dst_hbm, w_hbm, out_hbm,
                acc, sidx_v, didx_v, w_v, rows_v, zbuf, sem):
    cid = lax.axis_index("c")
    sid = lax.axis_index("s")
    wid = cid * NS + sid

    @pl.when(sid < zt)
    def _():
      def zfill(j, _):
        for c in range(n_chunks):
          zbuf[j, pl.ds(c * L, L)] = jnp.zeros((L,), jnp.float32)
        return 0
      lax.fori_loop(0, zr, zfill, 0)

      def zcopy(j, _):
        pltpu.sync_copy(zbuf, acc.at[pl.ds(sid * rows_per_tile + j * zr, zr)])
        return 0
      lax.fori_loop(0, rows_per_tile // zr, zcopy, 0)
    plsc.subcore_barrier()

    ebase = wid * epw

    def edge_batch(b, _):
      off = ebase + b * EDGE_BATCH
      pltpu.sync_copy(src_hbm.at[pl.ds(off, EDGE_BATCH)], sidx_v)
      pltpu.sync_copy(dst_hbm.at[pl.ds(off, EDGE_BATCH)], didx_v)
      pltpu.sync_copy(w_hbm.at[pl.ds(off, EDGE_BATCH)], w_v)
      pltpu.async_copy(x_hbm.at[sidx_v], rows_v, sem).wait()

      def scale_group(g, _):
        wvec = w_v[pl.ds(g * L, L)]
        for jj in range(L):
          wv = jnp.full((L,), wvec[jj], jnp.float32)
          j = g * L + jj
          for c in range(n_chunks):
            rows_v[j, pl.ds(c * L, L)] = rows_v[j, pl.ds(c * L, L)] * wv
        return 0
      lax.fori_loop(0, EDGE_BATCH // L, scale_group, 0)

      pltpu.sync_copy(rows_v, acc.at[didx_v], add=True)
      return 0
    lax.fori_loop(0, n_batches, edge_batch, 0)
    plsc.subcore_barrier()

    @pl.when(sid < zt)
    def _():
      pltpu.sync_copy(
          acc.at[pl.ds(sid * rows_per_tile, rows_per_tile)],
          out_hbm.at[cid, pl.ds(sid * rows_per_tile, rows_per_tile)])

  return sc_kernel


def _combine_body(p_ref, o_ref):
  o_ref[...] = p_ref[0] + p_ref[1]


@functools.lru_cache(maxsize=None)
def _build_combine(n_nodes, d_feat):
  grid = 10 if n_nodes % 80 == 0 else 1
  blk = n_nodes // grid
  return pl.pallas_call(
      _combine_body,
      grid=(grid,),
      in_specs=[pl.BlockSpec((NC, blk, d_feat), lambda i: (0, i, 0))],
      out_specs=pl.BlockSpec((blk, d_feat), lambda i: (i, 0)),
      out_shape=jax.ShapeDtypeStruct((n_nodes, d_feat), jnp.float32),
  )


def kernel(x, edge_index, edge_weight):
  n_nodes, d_feat = x.shape
  n_edges = edge_index.shape[1]
  src = edge_index[0].astype(jnp.int32)
  dst = edge_index[1].astype(jnp.int32)
  w = edge_weight.astype(jnp.float32)

  chunk = NW * EDGE_BATCH
  e_pad = ((n_edges + chunk - 1) // chunk) * chunk
  if e_pad != n_edges:
    pad = e_pad - n_edges
    src = jnp.concatenate([src, jnp.zeros((pad,), jnp.int32)])
    dst = jnp.concatenate([dst, jnp.zeros((pad,), jnp.int32)])
    w = jnp.concatenate([w, jnp.zeros((pad,), jnp.float32)])

  partial = _build_sc_kernel(n_nodes, d_feat, e_pad)(x, src, dst, w)
  return _build_combine(n_nodes, d_feat)(partial)
